# Initial kernel scaffold; baseline (speedup 1.0000x reference)
import jax
import jax.numpy as jnp
from jax.experimental import pallas as pl


def kernel(x, edge_index, W0, b0, W1, b1, W2, b2, Wmu, bmu, Wls, bls):
    raise NotImplementedError("write your pallas kernel here")


# speedup vs baseline: 16.8150x; 16.8150x over previous
import functools

import jax
import jax.numpy as jnp
from jax import lax
from jax.experimental import pallas as pl
from jax.experimental.pallas import tpu as pltpu
from jax.experimental.pallas import tpu_sc as plsc

N = 10000
E = 320000
D = 128
NC, NS = 2, 16
NW = NC * NS
EPW = E // NW
EB = 80
NB = EPW // EB
NP = 10240
RPT = NP // NS
RP = N // NS

_mesh = plsc.VectorSubcoreMesh(
    core_axis_name="c", subcore_axis_name="s", num_cores=NC, num_subcores=NS)


def _sc_degree(dst_r, ones_hbm):
  @functools.partial(
      pl.kernel,
      out_type=jax.ShapeDtypeStruct((NC, NP, 1), jnp.float32),
      mesh=_mesh,
      scratch_types=[
          pltpu.VMEM((NB, EB), jnp.int32),
          pltpu.VMEM((EB, 1), jnp.float32),
          pltpu.VMEM_SHARED((NP, 1), jnp.float32),
      ],
  )
  def k(dst_hbm, ones_h, out_hbm, didx, onesv, dacc):
    c = lax.axis_index("c")
    s = lax.axis_index("s")
    wid = c * NS + s
    pltpu.sync_copy(dst_hbm.at[wid], didx)
    pltpu.sync_copy(ones_h.at[pl.ds(0, EB)], onesv)
    pltpu.sync_copy(ones_h.at[pl.ds(s * RPT, RPT)], dacc.at[pl.ds(s * RPT, RPT)])
    plsc.subcore_barrier()

    def body(j, carry):
      pltpu.sync_copy(onesv, dacc.at[didx.at[j]], add=True)
      return carry

    lax.fori_loop(0, NB, body, 0)
    plsc.subcore_barrier()
    pltpu.sync_copy(dacc.at[pl.ds(s * RPT, RPT)],
                    out_hbm.at[c, pl.ds(s * RPT, RPT)])

  return k(dst_r, ones_hbm)


def _sc_adj(u, src_r, dst_r):
  @functools.partial(
      pl.kernel,
      out_type=jax.ShapeDtypeStruct((NC, NP, D), jnp.float32),
      mesh=_mesh,
      scratch_types=[
          pltpu.VMEM((NB, EB), jnp.int32),
          pltpu.VMEM((NB, EB), jnp.int32),
          pltpu.VMEM((EB, D), jnp.float32),
          pltpu.VMEM_SHARED((NP, D), jnp.float32),
          pltpu.SemaphoreType.DMA,
      ],
  )
  def k(u_hbm, src_hbm, dst_hbm, out_hbm, sidx, didx, rows, acc, gsem):
    c = lax.axis_index("c")
    s = lax.axis_index("s")
    wid = c * NS + s
    pltpu.sync_copy(src_hbm.at[wid], sidx)
    pltpu.sync_copy(dst_hbm.at[wid], didx)
    pltpu.sync_copy(u_hbm.at[pl.ds(s * RPT, RPT)], acc.at[pl.ds(s * RPT, RPT)])
    plsc.subcore_barrier()

    def body(j, carry):
      pltpu.async_copy(u_hbm.at[sidx.at[j]], rows, gsem).wait()
      pltpu.sync_copy(rows, acc.at[didx.at[j]], add=True)
      return carry

    lax.fori_loop(0, NB, body, 0)
    plsc.subcore_barrier()
    pltpu.sync_copy(acc.at[pl.ds(s * RPT, RPT)],
                    out_hbm.at[c, pl.ds(s * RPT, RPT)])

  return k(u, src_r, dst_r)


def _tc_scale(x, dinv):
  def body(x_ref, d_ref, o_ref):
    o_ref[...] = x_ref[...] * d_ref[...]

  return pl.pallas_call(
      body, out_shape=jax.ShapeDtypeStruct((NP, D), jnp.float32))(x, dinv)


def _tc_layer(acc, u_prev, dinv, w, b):
  def body(a_ref, u_ref, d_ref, w_ref, b_ref, o_ref):
    g = d_ref[...] * (a_ref[0] + a_ref[1] - u_ref[...])
    h = jnp.dot(g, w_ref[...], preferred_element_type=jnp.float32) + b_ref[...]
    o_ref[...] = d_ref[...] * jnp.maximum(h, 0.0)

  return pl.pallas_call(
      body, out_shape=jax.ShapeDtypeStruct((NP, D), jnp.float32))(
          acc, u_prev, dinv, w, b.reshape(1, D))


def _tc_head(acc, u_prev, dinv, wmu, bmu, wls, bls):
  dout = wmu.shape[1]

  def body(a_ref, u_ref, d_ref, wm_ref, bm_ref, wl_ref, bl_ref,
           mu_ref, ls_ref):
    g = d_ref[...] * (a_ref[0] + a_ref[1] - u_ref[...])
    mu_ref[...] = jnp.dot(
        g, wm_ref[...], preferred_element_type=jnp.float32) + bm_ref[...]
    ls_ref[...] = jnp.dot(
        g, wl_ref[...], preferred_element_type=jnp.float32) + bl_ref[...]

  return pl.pallas_call(
      body,
      out_shape=(jax.ShapeDtypeStruct((NP, dout), jnp.float32),
                 jax.ShapeDtypeStruct((NP, dout), jnp.float32)))(
          acc, u_prev, dinv, wmu, bmu.reshape(1, dout),
          wls, bls.reshape(1, dout))


def kernel(x, edge_index, W0, b0, W1, b1, W2, b2, Wmu, bmu, Wls, bls):
  ei = edge_index.astype(jnp.int32)
  src_r = ei[0].reshape(NW, NB, EB)
  dst_r = ei[1].reshape(NW, NB, EB)
  ones_hbm = jnp.ones((NP, 1), jnp.float32)
  x_p = jnp.pad(x, ((0, NP - N), (0, 0)))

  deg_p = _sc_degree(dst_r, ones_hbm)
  deg = deg_p[0, :, 0] + deg_p[1, :, 0] - 1.0
  dinv = lax.rsqrt(deg)[:, None]

  u = _tc_scale(x_p, dinv)
  for w, b in ((W0, b0), (W1, b1), (W2, b2)):
    acc = _sc_adj(u, src_r, dst_r)
    u = _tc_layer(acc, u, dinv, w, b)
  acc = _sc_adj(u, src_r, dst_r)
  mu_p, ls_p = _tc_head(acc, u, dinv, Wmu, bmu, Wls, bls)
  return (mu_p[:N], ls_p[:N])


# speedup vs baseline: 28.7360x; 1.7089x over previous
import functools

import jax
import jax.numpy as jnp
from jax import lax
from jax.experimental import pallas as pl
from jax.experimental.pallas import tpu as pltpu
from jax.experimental.pallas import tpu_sc as plsc

N = 10000
E = 320000
D = 128
NC, NS = 2, 16
NW = NC * NS
NP = 10240
RPT = NP // NS
EB = 128
RB = 8
PH = 2
CHP = 5
NBP = CHP * RB
EPWP = PH * NBP * EB
EPW = E // NW

_mesh = plsc.VectorSubcoreMesh(
    core_axis_name="c", subcore_axis_name="s", num_cores=NC, num_subcores=NS)


def _sc_degree(dst_r, ones_hbm):
  @functools.partial(
      pl.kernel,
      out_type=jax.ShapeDtypeStruct((NC, NP, 1), jnp.float32),
      mesh=_mesh,
      scratch_types=[
          pltpu.VMEM((CHP, RB, EB), jnp.int32),
          pltpu.VMEM((EB, 1), jnp.float32),
          pltpu.VMEM_SHARED((NP, 1), jnp.float32),
      ],
  )
  def k(dst_hbm, ones_h, out_hbm, didx, onesv, dacc):
    c = lax.axis_index("c")
    s = lax.axis_index("s")
    wid = c * NS + s
    pltpu.sync_copy(ones_h.at[pl.ds(0, EB)], onesv)
    pltpu.sync_copy(ones_h.at[pl.ds(s * RPT, RPT)], dacc.at[pl.ds(s * RPT, RPT)])
    plsc.subcore_barrier()
    for ph in range(PH):
      pltpu.sync_copy(dst_hbm.at[wid, ph], didx)

      def body(j, carry):
        pltpu.sync_copy(onesv, dacc.at[didx.at[j // RB, lax.rem(j, RB)]],
                        add=True)
        return carry

      lax.fori_loop(0, NBP, body, 0)
    plsc.subcore_barrier()
    pltpu.sync_copy(dacc.at[pl.ds(s * RPT, RPT)],
                    out_hbm.at[c, pl.ds(s * RPT, RPT)])

  return k(dst_r, ones_hbm)


def _sc_adj(u, src_r, dst_r):
  @functools.partial(
      pl.kernel,
      out_type=jax.ShapeDtypeStruct((NC, NP, D), jnp.float32),
      mesh=_mesh,
      scratch_types=[
          pltpu.VMEM((CHP, RB, EB), jnp.int32),
          pltpu.VMEM((CHP, RB, EB), jnp.int32),
          pltpu.VMEM((2, EB, D), jnp.float32),
          pltpu.VMEM_SHARED((NP, D), jnp.float32),
          pltpu.SemaphoreType.DMA,
          pltpu.SemaphoreType.DMA,
      ],
  )
  def k(u_hbm, src_hbm, dst_hbm, out_hbm, sidx, didx, rows, acc, gsem, ssem):
    c = lax.axis_index("c")
    s = lax.axis_index("s")
    wid = c * NS + s
    pltpu.sync_copy(u_hbm.at[pl.ds(s * RPT, RPT)], acc.at[pl.ds(s * RPT, RPT)])
    plsc.subcore_barrier()

    def sref(j):
      return sidx.at[j // RB, lax.rem(j, RB)]

    def dref(j):
      return didx.at[j // RB, lax.rem(j, RB)]

    def _wait_gather(slot, j):
      pltpu.make_async_copy(u_hbm.at[sref(j)], rows.at[slot], gsem).wait()

    def _drain_scatter(slot, j):
      pltpu.make_async_copy(rows.at[slot], acc.at[dref(j)], ssem).wait()

    for ph in range(PH):
      pltpu.sync_copy(src_hbm.at[wid, ph], sidx)
      pltpu.sync_copy(dst_hbm.at[wid, ph], didx)
      pltpu.async_copy(u_hbm.at[sref(0)], rows.at[0], gsem)

      def body(i, carry):
        j0 = 2 * i
        j1 = j0 + 1

        @pl.when(i > 0)
        def _():
          _drain_scatter(1, j0)
        pltpu.async_copy(u_hbm.at[sref(j1)], rows.at[1], gsem)
        _wait_gather(0, j0)
        pltpu.async_copy(rows.at[0], acc.at[dref(j0)], ssem)
        _wait_gather(1, j1)
        _drain_scatter(0, j0)

        @pl.when(j0 + 2 < NBP)
        def _():
          pltpu.async_copy(u_hbm.at[sref(j0 + 2)], rows.at[0], gsem)
        pltpu.async_copy(rows.at[1], acc.at[dref(j1)], ssem)
        return carry

      lax.fori_loop(0, NBP // 2, body, 0)
      _drain_scatter(1, 0)
    plsc.subcore_barrier()
    pltpu.sync_copy(acc.at[pl.ds(s * RPT, RPT)],
                    out_hbm.at[c, pl.ds(s * RPT, RPT)])

  return k(u, src_r, dst_r)


def _tc_scale(x, dinv):
  def body(x_ref, d_ref, o_ref):
    o_ref[...] = x_ref[...] * d_ref[...]

  return pl.pallas_call(
      body, out_shape=jax.ShapeDtypeStruct((NP, D), jnp.float32))(x, dinv)


def _tc_layer(acc, u_prev, dinv, w, b):
  def body(a_ref, u_ref, d_ref, w_ref, b_ref, o_ref):
    g = d_ref[...] * (a_ref[0] + a_ref[1] - u_ref[...])
    h = jnp.dot(g, w_ref[...], preferred_element_type=jnp.float32) + b_ref[...]
    o_ref[...] = d_ref[...] * jnp.maximum(h, 0.0)

  return pl.pallas_call(
      body, out_shape=jax.ShapeDtypeStruct((NP, D), jnp.float32))(
          acc, u_prev, dinv, w, b.reshape(1, D))


def _tc_head(acc, u_prev, dinv, wmu, bmu, wls, bls):
  dout = wmu.shape[1]

  def body(a_ref, u_ref, d_ref, wm_ref, bm_ref, wl_ref, bl_ref,
           mu_ref, ls_ref):
    g = d_ref[...] * (a_ref[0] + a_ref[1] - u_ref[...])
    mu_ref[...] = jnp.dot(
        g, wm_ref[...], preferred_element_type=jnp.float32) + bm_ref[...]
    ls_ref[...] = jnp.dot(
        g, wl_ref[...], preferred_element_type=jnp.float32) + bl_ref[...]

  return pl.pallas_call(
      body,
      out_shape=(jax.ShapeDtypeStruct((NP, dout), jnp.float32),
                 jax.ShapeDtypeStruct((NP, dout), jnp.float32)))(
          acc, u_prev, dinv, wmu, bmu.reshape(1, dout),
          wls, bls.reshape(1, dout))


def _pad_edges(idx):
  w = idx.reshape(NW, EPW)
  pad = jnp.broadcast_to(
      jnp.arange(N, N + (EPWP - EPW), dtype=jnp.int32)[None, :],
      (NW, EPWP - EPW))
  return jnp.concatenate([w, pad], axis=1).reshape(NW, PH, CHP, RB, EB)


def kernel(x, edge_index, W0, b0, W1, b1, W2, b2, Wmu, bmu, Wls, bls):
  ei = edge_index.astype(jnp.int32)
  src_r = _pad_edges(ei[0])
  dst_r = _pad_edges(ei[1])
  ones_hbm = jnp.ones((NP, 1), jnp.float32)
  x_p = jnp.pad(x, ((0, NP - N), (0, 0)))

  deg_p = _sc_degree(dst_r, ones_hbm)
  deg = deg_p[0, :, 0] + deg_p[1, :, 0] - 1.0
  dinv = lax.rsqrt(deg)[:, None]

  u = _tc_scale(x_p, dinv)
  for w, b in ((W0, b0), (W1, b1), (W2, b2)):
    acc = _sc_adj(u, src_r, dst_r)
    u = _tc_layer(acc, u, dinv, w, b)
  acc = _sc_adj(u, src_r, dst_r)
  mu_p, ls_p = _tc_head(acc, u, dinv, Wmu, bmu, Wls, bls)
  return (mu_p[:N], ls_p[:N])


# speedup vs baseline: 29.0659x; 1.0115x over previous
import functools

import jax
import jax.numpy as jnp
from jax import lax
from jax.experimental import pallas as pl
from jax.experimental.pallas import tpu as pltpu
from jax.experimental.pallas import tpu_sc as plsc

N = 10000
E = 320000
D = 128
NC, NS = 2, 16
NW = NC * NS
NP = 10240
RPT = NP // NS
EB = 128
RB = 8
PH = 2
CHP = 5
NBP = CHP * RB
EPWP = PH * NBP * EB
EPW = E // NW

_mesh = plsc.VectorSubcoreMesh(
    core_axis_name="c", subcore_axis_name="s", num_cores=NC, num_subcores=NS)


def _sc_degree(dst_r, ones_hbm):
  @functools.partial(
      pl.kernel,
      out_type=jax.ShapeDtypeStruct((NC, NP, 1), jnp.float32),
      mesh=_mesh,
      scratch_types=[
          pltpu.VMEM((CHP, RB, EB), jnp.int32),
          pltpu.VMEM((EB, 1), jnp.float32),
          pltpu.VMEM_SHARED((NP, 1), jnp.float32),
          pltpu.SemaphoreType.DMA,
      ],
  )
  def k(dst_hbm, ones_h, out_hbm, didx, onesv, dacc, dsem):
    c = lax.axis_index("c")
    s = lax.axis_index("s")
    wid = c * NS + s
    pltpu.sync_copy(ones_h.at[pl.ds(0, EB)], onesv)
    pltpu.sync_copy(ones_h.at[pl.ds(s * RPT, RPT)], dacc.at[pl.ds(s * RPT, RPT)])
    plsc.subcore_barrier()
    for ph in range(PH):
      pltpu.sync_copy(dst_hbm.at[wid, ph], didx)

      def body(j, carry):
        pltpu.async_copy(onesv, dacc.at[didx.at[j // RB, lax.rem(j, RB)]],
                         dsem)
        return carry

      lax.fori_loop(0, NBP, body, 0)

      def drain(j, carry):
        pltpu.make_async_copy(onesv, dacc.at[didx.at[0, 0]], dsem).wait()
        return carry

      lax.fori_loop(0, NBP, drain, 0)
    plsc.subcore_barrier()
    pltpu.sync_copy(dacc.at[pl.ds(s * RPT, RPT)],
                    out_hbm.at[c, pl.ds(s * RPT, RPT)])

  return k(dst_r, ones_hbm)


def _sc_adj(u, src_r, dst_r):
  @functools.partial(
      pl.kernel,
      out_type=jax.ShapeDtypeStruct((NC, NP, D), jnp.float32),
      mesh=_mesh,
      scratch_types=[
          pltpu.VMEM((CHP, RB, EB), jnp.int32),
          pltpu.VMEM((CHP, RB, EB), jnp.int32),
          pltpu.VMEM((2, EB, D), jnp.float32),
          pltpu.VMEM_SHARED((NP, D), jnp.float32),
          pltpu.SemaphoreType.DMA,
          pltpu.SemaphoreType.DMA,
      ],
  )
  def k(u_hbm, src_hbm, dst_hbm, out_hbm, sidx, didx, rows, acc, gsem, ssem):
    c = lax.axis_index("c")
    s = lax.axis_index("s")
    wid = c * NS + s
    pltpu.sync_copy(u_hbm.at[pl.ds(s * RPT, RPT)], acc.at[pl.ds(s * RPT, RPT)])
    plsc.subcore_barrier()

    def sref(j):
      return sidx.at[j // RB, lax.rem(j, RB)]

    def dref(j):
      return didx.at[j // RB, lax.rem(j, RB)]

    def _wait_gather(slot, j):
      pltpu.make_async_copy(u_hbm.at[sref(j)], rows.at[slot], gsem).wait()

    def _drain_scatter(slot, j):
      pltpu.make_async_copy(rows.at[slot], acc.at[dref(j)], ssem).wait()

    for ph in range(PH):
      pltpu.sync_copy(src_hbm.at[wid, ph], sidx)
      pltpu.sync_copy(dst_hbm.at[wid, ph], didx)
      pltpu.async_copy(u_hbm.at[sref(0)], rows.at[0], gsem)

      def body(i, carry):
        j0 = 2 * i
        j1 = j0 + 1

        @pl.when(i > 0)
        def _():
          _drain_scatter(1, j0)
        pltpu.async_copy(u_hbm.at[sref(j1)], rows.at[1], gsem)
        _wait_gather(0, j0)
        pltpu.async_copy(rows.at[0], acc.at[dref(j0)], ssem)
        _wait_gather(1, j1)
        _drain_scatter(0, j0)

        @pl.when(j0 + 2 < NBP)
        def _():
          pltpu.async_copy(u_hbm.at[sref(j0 + 2)], rows.at[0], gsem)
        pltpu.async_copy(rows.at[1], acc.at[dref(j1)], ssem)
        return carry

      lax.fori_loop(0, NBP // 2, body, 0)
      _drain_scatter(1, 0)
    plsc.subcore_barrier()
    pltpu.sync_copy(acc.at[pl.ds(s * RPT, RPT)],
                    out_hbm.at[c, pl.ds(s * RPT, RPT)])

  return k(u, src_r, dst_r)


def _tc_matmul(x, w):
  def body(x_ref, w_ref, o_ref):
    o_ref[...] = jnp.dot(x_ref[...], w_ref[...],
                         preferred_element_type=jnp.float32)

  return pl.pallas_call(
      body, out_shape=jax.ShapeDtypeStruct((NP, D), jnp.float32))(x, w)


def _tc_scale(deg_p, y):
  def body(dp_ref, y_ref, d_ref, u_ref):
    dinv = lax.rsqrt(dp_ref[0] + dp_ref[1] - 1.0)
    d_ref[...] = dinv
    u_ref[...] = y_ref[...] * dinv

  return pl.pallas_call(
      body, out_shape=(jax.ShapeDtypeStruct((NP, 1), jnp.float32),
                       jax.ShapeDtypeStruct((NP, D), jnp.float32)))(deg_p, y)


def _tc_layer0(acc, u_prev, dinv, b):
  def body(a_ref, u_ref, d_ref, b_ref, o_ref):
    g = d_ref[...] * (a_ref[0] + a_ref[1] - u_ref[...])
    o_ref[...] = d_ref[...] * jnp.maximum(g + b_ref[...], 0.0)

  return pl.pallas_call(
      body, out_shape=jax.ShapeDtypeStruct((NP, D), jnp.float32))(
          acc, u_prev, dinv, b.reshape(1, D))


def _tc_layer(acc, u_prev, dinv, w, b):
  def body(a_ref, u_ref, d_ref, w_ref, b_ref, o_ref):
    g = d_ref[...] * (a_ref[0] + a_ref[1] - u_ref[...])
    h = jnp.dot(g, w_ref[...], preferred_element_type=jnp.float32) + b_ref[...]
    o_ref[...] = d_ref[...] * jnp.maximum(h, 0.0)

  return pl.pallas_call(
      body, out_shape=jax.ShapeDtypeStruct((NP, D), jnp.float32))(
          acc, u_prev, dinv, w, b.reshape(1, D))


def _tc_head(acc, u_prev, dinv, wmu, bmu, wls, bls):
  dout = wmu.shape[1]

  def body(a_ref, u_ref, d_ref, wm_ref, bm_ref, wl_ref, bl_ref,
           mu_ref, ls_ref):
    g = d_ref[...] * (a_ref[0] + a_ref[1] - u_ref[...])
    mu_ref[...] = jnp.dot(
        g, wm_ref[...], preferred_element_type=jnp.float32) + bm_ref[...]
    ls_ref[...] = jnp.dot(
        g, wl_ref[...], preferred_element_type=jnp.float32) + bl_ref[...]

  return pl.pallas_call(
      body,
      out_shape=(jax.ShapeDtypeStruct((NP, dout), jnp.float32),
                 jax.ShapeDtypeStruct((NP, dout), jnp.float32)))(
          acc, u_prev, dinv, wmu, bmu.reshape(1, dout),
          wls, bls.reshape(1, dout))


def _pad_edges(idx):
  w = idx.reshape(NW, EPW)
  pad = jnp.broadcast_to(
      jnp.arange(N, N + (EPWP - EPW), dtype=jnp.int32)[None, :],
      (NW, EPWP - EPW))
  return jnp.concatenate([w, pad], axis=1).reshape(NW, PH, CHP, RB, EB)


def kernel(x, edge_index, W0, b0, W1, b1, W2, b2, Wmu, bmu, Wls, bls):
  ei = edge_index.astype(jnp.int32)
  src_r = _pad_edges(ei[0])
  dst_r = _pad_edges(ei[1])
  ones_hbm = jnp.ones((NP, 1), jnp.float32)
  x_p = jnp.pad(x, ((0, NP - N), (0, 0)))

  y0 = _tc_matmul(x_p, W0)
  deg_p = _sc_degree(dst_r, ones_hbm)
  dinv, u = _tc_scale(deg_p, y0)

  acc = _sc_adj(u, src_r, dst_r)
  u = _tc_layer0(acc, u, dinv, b0)
  for w, b in ((W1, b1), (W2, b2)):
    acc = _sc_adj(u, src_r, dst_r)
    u = _tc_layer(acc, u, dinv, w, b)
  acc = _sc_adj(u, src_r, dst_r)
  mu_p, ls_p = _tc_head(acc, u, dinv, Wmu, bmu, Wls, bls)
  return (mu_p[:N], ls_p[:N])


# speedup vs baseline: 29.8285x; 1.0262x over previous
import functools

import jax
import jax.numpy as jnp
from jax import lax
from jax.experimental import pallas as pl
from jax.experimental.pallas import tpu as pltpu
from jax.experimental.pallas import tpu_sc as plsc

N = 10000
E = 320000
D = 128
NC, NS = 2, 16
NW = NC * NS
NP = 10240
RPT = NP // NS
EB = 128
RB = 8
PH = 2
CHP = 5
NBP = CHP * RB
EPWP = PH * NBP * EB
EPW = E // NW

_mesh = plsc.VectorSubcoreMesh(
    core_axis_name="c", subcore_axis_name="s", num_cores=NC, num_subcores=NS)


def _sc_degree(dst_r, ones_hbm):
  @functools.partial(
      pl.kernel,
      out_type=jax.ShapeDtypeStruct((NC, NP, 1), jnp.float32),
      mesh=_mesh,
      scratch_types=[
          pltpu.VMEM((CHP, RB, EB), jnp.int32),
          pltpu.VMEM((EB, 1), jnp.float32),
          pltpu.VMEM_SHARED((NP, 1), jnp.float32),
          pltpu.SemaphoreType.DMA,
      ],
  )
  def k(dst_hbm, ones_h, out_hbm, didx, onesv, dacc, dsem):
    c = lax.axis_index("c")
    s = lax.axis_index("s")
    wid = c * NS + s
    pltpu.sync_copy(ones_h.at[pl.ds(0, EB)], onesv)
    pltpu.sync_copy(ones_h.at[pl.ds(s * RPT, RPT)], dacc.at[pl.ds(s * RPT, RPT)])
    plsc.subcore_barrier()
    for ph in range(PH):
      pltpu.sync_copy(dst_hbm.at[wid, ph], didx)

      def body(j, carry):
        pltpu.async_copy(onesv, dacc.at[didx.at[j // RB, lax.rem(j, RB)]],
                         dsem)
        return carry

      lax.fori_loop(0, NBP, body, 0)

      def drain(j, carry):
        pltpu.make_async_copy(onesv, dacc.at[didx.at[0, 0]], dsem).wait()
        return carry

      lax.fori_loop(0, NBP, drain, 0)
    plsc.subcore_barrier()
    pltpu.sync_copy(dacc.at[pl.ds(s * RPT, RPT)],
                    out_hbm.at[c, pl.ds(s * RPT, RPT)])

  return k(dst_r, ones_hbm)


def _sc_adj(u, src_r, dst_r):
  @functools.partial(
      pl.kernel,
      out_type=jax.ShapeDtypeStruct((NC, NP, D), jnp.float32),
      mesh=_mesh,
      scratch_types=[
          pltpu.VMEM((CHP, RB, EB), jnp.int32),
          pltpu.VMEM((CHP, RB, EB), jnp.int32),
          pltpu.VMEM((2, EB, D), jnp.float32),
          pltpu.VMEM_SHARED((NP, D), jnp.float32),
          pltpu.SemaphoreType.DMA,
          pltpu.SemaphoreType.DMA,
          pltpu.SemaphoreType.DMA,
      ],
  )
  def k(u_hbm, src_hbm, dst_hbm, out_hbm, sidx, didx, rows, acc, gsem, ssem,
        isem):
    c = lax.axis_index("c")
    s = lax.axis_index("s")
    wid = c * NS + s
    init = pltpu.async_copy(
        u_hbm.at[pl.ds(s * RPT, RPT)], acc.at[pl.ds(s * RPT, RPT)], isem)

    def sref(j):
      return sidx.at[j // RB, lax.rem(j, RB)]

    def dref(j):
      return didx.at[j // RB, lax.rem(j, RB)]

    def _wait_gather(slot, j):
      pltpu.make_async_copy(u_hbm.at[sref(j)], rows.at[slot], gsem).wait()

    def _drain_scatter(slot, j):
      pltpu.make_async_copy(rows.at[slot], acc.at[dref(j)], ssem).wait()

    for ph in range(PH):
      pltpu.sync_copy(src_hbm.at[wid, ph], sidx)
      pltpu.sync_copy(dst_hbm.at[wid, ph], didx)
      pltpu.async_copy(u_hbm.at[sref(0)], rows.at[0], gsem)
      if ph == 0:
        init.wait()
        plsc.subcore_barrier()

      def body(i, carry):
        j0 = 2 * i
        j1 = j0 + 1

        @pl.when(i > 0)
        def _():
          _drain_scatter(1, j0)
        pltpu.async_copy(u_hbm.at[sref(j1)], rows.at[1], gsem)
        _wait_gather(0, j0)
        pltpu.async_copy(rows.at[0], acc.at[dref(j0)], ssem)
        _wait_gather(1, j1)
        _drain_scatter(0, j0)

        @pl.when(j0 + 2 < NBP)
        def _():
          pltpu.async_copy(u_hbm.at[sref(j0 + 2)], rows.at[0], gsem)
        pltpu.async_copy(rows.at[1], acc.at[dref(j1)], ssem)
        return carry

      lax.fori_loop(0, NBP // 2, body, 0)
      _drain_scatter(1, 0)
    plsc.subcore_barrier()
    pltpu.sync_copy(acc.at[pl.ds(s * RPT, RPT)],
                    out_hbm.at[c, pl.ds(s * RPT, RPT)])

  return k(u, src_r, dst_r)


def _tc_matmul(x, w):
  def body(x_ref, w_ref, o_ref):
    o_ref[...] = jnp.dot(x_ref[...], w_ref[...],
                         preferred_element_type=jnp.float32)

  return pl.pallas_call(
      body, out_shape=jax.ShapeDtypeStruct((NP, D), jnp.float32))(x, w)


def _tc_scale(deg_p, y):
  def body(dp_ref, y_ref, d_ref, u_ref):
    dinv = lax.rsqrt(dp_ref[0] + dp_ref[1] - 1.0)
    d_ref[...] = dinv
    u_ref[...] = y_ref[...] * dinv

  return pl.pallas_call(
      body, out_shape=(jax.ShapeDtypeStruct((NP, 1), jnp.float32),
                       jax.ShapeDtypeStruct((NP, D), jnp.float32)))(deg_p, y)


def _tc_layer0(acc, u_prev, dinv, b):
  def body(a_ref, u_ref, d_ref, b_ref, o_ref):
    g = d_ref[...] * (a_ref[0] + a_ref[1] - u_ref[...])
    o_ref[...] = d_ref[...] * jnp.maximum(g + b_ref[...], 0.0)

  return pl.pallas_call(
      body, out_shape=jax.ShapeDtypeStruct((NP, D), jnp.float32))(
          acc, u_prev, dinv, b.reshape(1, D))


def _tc_layer(acc, u_prev, dinv, w, b):
  def body(a_ref, u_ref, d_ref, w_ref, b_ref, o_ref):
    g = d_ref[...] * (a_ref[0] + a_ref[1] - u_ref[...])
    h = jnp.dot(g, w_ref[...], preferred_element_type=jnp.float32) + b_ref[...]
    o_ref[...] = d_ref[...] * jnp.maximum(h, 0.0)

  return pl.pallas_call(
      body, out_shape=jax.ShapeDtypeStruct((NP, D), jnp.float32))(
          acc, u_prev, dinv, w, b.reshape(1, D))


def _tc_head(acc, u_prev, dinv, wmu, bmu, wls, bls):
  dout = wmu.shape[1]

  def body(a_ref, u_ref, d_ref, wm_ref, bm_ref, wl_ref, bl_ref,
           mu_ref, ls_ref):
    g = (d_ref[...] * (a_ref[0] + a_ref[1] - u_ref[...]))[:N]
    mu_ref[...] = jnp.dot(
        g, wm_ref[...], preferred_element_type=jnp.float32) + bm_ref[...]
    ls_ref[...] = jnp.dot(
        g, wl_ref[...], preferred_element_type=jnp.float32) + bl_ref[...]

  return pl.pallas_call(
      body,
      out_shape=(jax.ShapeDtypeStruct((N, dout), jnp.float32),
                 jax.ShapeDtypeStruct((N, dout), jnp.float32)))(
          acc, u_prev, dinv, wmu, bmu.reshape(1, dout),
          wls, bls.reshape(1, dout))


def _pad_edges(idx):
  w = idx.reshape(NW, EPW)
  pad = jnp.broadcast_to(
      jnp.arange(N, N + (EPWP - EPW), dtype=jnp.int32)[None, :],
      (NW, EPWP - EPW))
  return jnp.concatenate([w, pad], axis=1).reshape(NW, PH, CHP, RB, EB)


def kernel(x, edge_index, W0, b0, W1, b1, W2, b2, Wmu, bmu, Wls, bls):
  ei = edge_index.astype(jnp.int32)
  src_r = _pad_edges(ei[0])
  dst_r = _pad_edges(ei[1])
  ones_hbm = jnp.ones((NP, 1), jnp.float32)
  x_p = jnp.pad(x, ((0, NP - N), (0, 0)))

  y0 = _tc_matmul(x_p, W0)
  deg_p = _sc_degree(dst_r, ones_hbm)
  dinv, u = _tc_scale(deg_p, y0)

  acc = _sc_adj(u, src_r, dst_r)
  u = _tc_layer0(acc, u, dinv, b0)
  for w, b in ((W1, b1), (W2, b2)):
    acc = _sc_adj(u, src_r, dst_r)
    u = _tc_layer(acc, u, dinv, w, b)
  acc = _sc_adj(u, src_r, dst_r)
  mu, logstd = _tc_head(acc, u, dinv, Wmu, bmu, Wls, bls)
  return (mu, logstd)


# speedup vs baseline: 30.4021x; 1.0192x over previous
import functools

import jax
import jax.numpy as jnp
from jax import lax
from jax.experimental import pallas as pl
from jax.experimental.pallas import tpu as pltpu
from jax.experimental.pallas import tpu_sc as plsc

N = 10000
E = 320000
D = 128
NC, NS = 2, 16
NW = NC * NS
NP = 10240
RPT = NP // NS
EB = 128
RB = 8
PH = 2
CHP = 5
NBP = CHP * RB
EPWP = PH * NBP * EB
EPW = E // NW

_mesh = plsc.VectorSubcoreMesh(
    core_axis_name="c", subcore_axis_name="s", num_cores=NC, num_subcores=NS)


def _sc_degree(dst_r, ones_hbm):
  @functools.partial(
      pl.kernel,
      out_type=jax.ShapeDtypeStruct((NC, NP, 1), jnp.float32),
      mesh=_mesh,
      scratch_types=[
          pltpu.VMEM((CHP, RB, EB), jnp.int32),
          pltpu.VMEM((EB, 1), jnp.float32),
          pltpu.VMEM_SHARED((NP, 1), jnp.float32),
          pltpu.SemaphoreType.DMA,
      ],
  )
  def k(dst_hbm, ones_h, out_hbm, didx, onesv, dacc, dsem):
    c = lax.axis_index("c")
    s = lax.axis_index("s")
    wid = c * NS + s
    pltpu.sync_copy(ones_h.at[pl.ds(0, EB)], onesv)
    pltpu.sync_copy(ones_h.at[pl.ds(s * RPT, RPT)], dacc.at[pl.ds(s * RPT, RPT)])
    plsc.subcore_barrier()
    for ph in range(PH):
      pltpu.sync_copy(dst_hbm.at[wid, ph], didx)

      def body(j, carry):
        pltpu.async_copy(onesv, dacc.at[didx.at[j // RB, lax.rem(j, RB)]],
                         dsem)
        return carry

      lax.fori_loop(0, NBP, body, 0)

      def drain(j, carry):
        pltpu.make_async_copy(onesv, dacc.at[didx.at[0, 0]], dsem).wait()
        return carry

      lax.fori_loop(0, NBP, drain, 0)
    plsc.subcore_barrier()
    pltpu.sync_copy(dacc.at[pl.ds(s * RPT, RPT)],
                    out_hbm.at[c, pl.ds(s * RPT, RPT)])

  return k(dst_r, ones_hbm)


def _sc_adj(u, src_r, dst_r):
  @functools.partial(
      pl.kernel,
      out_type=jax.ShapeDtypeStruct((NC, NP, D), jnp.float32),
      mesh=_mesh,
      scratch_types=[
          pltpu.VMEM((CHP, RB, EB), jnp.int32),
          pltpu.VMEM((CHP, RB, EB), jnp.int32),
          pltpu.VMEM((2, EB, D), jnp.float32),
          pltpu.VMEM_SHARED((NP, D), jnp.float32),
          pltpu.SemaphoreType.DMA,
          pltpu.SemaphoreType.DMA,
          pltpu.SemaphoreType.DMA,
      ],
  )
  def k(u_hbm, src_hbm, dst_hbm, out_hbm, sidx, didx, rows, acc, gsem, ssem,
        isem):
    c = lax.axis_index("c")
    s = lax.axis_index("s")
    wid = c * NS + s
    init = pltpu.async_copy(
        u_hbm.at[pl.ds(s * RPT, RPT)], acc.at[pl.ds(s * RPT, RPT)], isem)

    def sref(j):
      return sidx.at[j // RB, lax.rem(j, RB)]

    def dref(j):
      return didx.at[j // RB, lax.rem(j, RB)]

    def _wait_gather(slot, j):
      pltpu.make_async_copy(u_hbm.at[sref(j)], rows.at[slot], gsem).wait()

    def _drain_scatter(slot, j):
      pltpu.make_async_copy(rows.at[slot], acc.at[dref(j)], ssem).wait()

    for ph in range(PH):
      pltpu.sync_copy(src_hbm.at[wid, ph], sidx)
      pltpu.sync_copy(dst_hbm.at[wid, ph], didx)
      pltpu.async_copy(u_hbm.at[sref(0)], rows.at[0], gsem)
      if ph == 0:
        init.wait()
        plsc.subcore_barrier()

      def body(i, carry):
        j0 = 2 * i
        j1 = j0 + 1

        @pl.when(i > 0)
        def _():
          _drain_scatter(1, j0)
        pltpu.async_copy(u_hbm.at[sref(j1)], rows.at[1], gsem)
        _wait_gather(0, j0)
        pltpu.async_copy(rows.at[0], acc.at[dref(j0)], ssem)
        _wait_gather(1, j1)
        _drain_scatter(0, j0)

        @pl.when(j0 + 2 < NBP)
        def _():
          pltpu.async_copy(u_hbm.at[sref(j0 + 2)], rows.at[0], gsem)
        pltpu.async_copy(rows.at[1], acc.at[dref(j1)], ssem)
        return carry

      lax.fori_loop(0, NBP // 2, body, 0)
      _drain_scatter(1, 0)
    plsc.subcore_barrier()
    pltpu.sync_copy(acc.at[pl.ds(s * RPT, RPT)],
                    out_hbm.at[c, pl.ds(s * RPT, RPT)])

  return k(u, src_r, dst_r)


def _tc_matmul(x, w):
  def body(x_ref, w_ref, o_ref):
    o_ref[pl.ds(0, N), :] = jnp.dot(x_ref[...], w_ref[...],
                                    preferred_element_type=jnp.float32)
    o_ref[pl.ds(N, NP - N), :] = jnp.zeros((NP - N, D), jnp.float32)

  return pl.pallas_call(
      body, out_shape=jax.ShapeDtypeStruct((NP, D), jnp.float32))(x, w)


def _tc_scale(deg_p, y):
  def body(dp_ref, y_ref, d_ref, u_ref):
    dinv = lax.rsqrt(dp_ref[0] + dp_ref[1] - 1.0)
    d_ref[...] = dinv
    u_ref[...] = y_ref[...] * dinv

  return pl.pallas_call(
      body, out_shape=(jax.ShapeDtypeStruct((NP, 1), jnp.float32),
                       jax.ShapeDtypeStruct((NP, D), jnp.float32)))(deg_p, y)


def _tc_layer0(acc, u_prev, dinv, b):
  def body(a_ref, u_ref, d_ref, b_ref, o_ref):
    g = d_ref[...] * (a_ref[0] + a_ref[1] - u_ref[...])
    o_ref[...] = d_ref[...] * jnp.maximum(g + b_ref[...], 0.0)

  return pl.pallas_call(
      body, out_shape=jax.ShapeDtypeStruct((NP, D), jnp.float32))(
          acc, u_prev, dinv, b.reshape(1, D))


def _tc_layer(acc, u_prev, dinv, w, b):
  def body(a_ref, u_ref, d_ref, w_ref, b_ref, o_ref):
    g = d_ref[...] * (a_ref[0] + a_ref[1] - u_ref[...])
    h = jnp.dot(g, w_ref[...], preferred_element_type=jnp.float32) + b_ref[...]
    o_ref[...] = d_ref[...] * jnp.maximum(h, 0.0)

  return pl.pallas_call(
      body, out_shape=jax.ShapeDtypeStruct((NP, D), jnp.float32))(
          acc, u_prev, dinv, w, b.reshape(1, D))


def _tc_head(acc, u_prev, dinv, wmu, bmu, wls, bls):
  dout = wmu.shape[1]

  def body(a_ref, u_ref, d_ref, wm_ref, bm_ref, wl_ref, bl_ref,
           mu_ref, ls_ref):
    g = (d_ref[...] * (a_ref[0] + a_ref[1] - u_ref[...]))[:N]
    mu_ref[...] = jnp.dot(
        g, wm_ref[...], preferred_element_type=jnp.float32) + bm_ref[...]
    ls_ref[...] = jnp.dot(
        g, wl_ref[...], preferred_element_type=jnp.float32) + bl_ref[...]

  return pl.pallas_call(
      body,
      out_shape=(jax.ShapeDtypeStruct((N, dout), jnp.float32),
                 jax.ShapeDtypeStruct((N, dout), jnp.float32)))(
          acc, u_prev, dinv, wmu, bmu.reshape(1, dout),
          wls, bls.reshape(1, dout))


def _pad_edges(ei):
  npad = NW * EPWP - E
  pad = jnp.broadcast_to(
      (N + jnp.arange(npad, dtype=jnp.int32) % (NP - N))[None, :], (2, npad))
  full = jnp.concatenate([ei, pad], axis=1).reshape(2, NW, PH, CHP, RB, EB)
  return full[0], full[1]


def kernel(x, edge_index, W0, b0, W1, b1, W2, b2, Wmu, bmu, Wls, bls):
  src_r, dst_r = _pad_edges(edge_index.astype(jnp.int32))
  ones_hbm = jnp.ones((NP, 1), jnp.float32)

  y0 = _tc_matmul(x, W0)
  deg_p = _sc_degree(dst_r, ones_hbm)
  dinv, u = _tc_scale(deg_p, y0)

  acc = _sc_adj(u, src_r, dst_r)
  u = _tc_layer0(acc, u, dinv, b0)
  for w, b in ((W1, b1), (W2, b2)):
    acc = _sc_adj(u, src_r, dst_r)
    u = _tc_layer(acc, u, dinv, w, b)
  acc = _sc_adj(u, src_r, dst_r)
  mu, logstd = _tc_head(acc, u, dinv, Wmu, bmu, Wls, bls)
  return (mu, logstd)
